# Initial kernel scaffold; baseline (speedup 1.0000x reference)
import jax
import jax.numpy as jnp
from jax.experimental import pallas as pl


def kernel(g_omics1, features_omics1, g_omics2, features_omics2, W1, att_src, att_dst, W2):
    raise NotImplementedError("write your pallas kernel here")


# speedup vs baseline: 14.8602x; 14.8602x over previous
import jax
import jax.numpy as jnp
from jax import lax
from jax.experimental import pallas as pl
from jax.experimental.pallas import tpu as pltpu
from jax.experimental.pallas import tpu_sc as plsc

N = 10000
NP = 10240
E = 320000
EP = 327680
IN_DIM, HID_DIM, OUT_DIM = 128, 512, 128
NTILES = 16
ET = EP // NTILES
BB = 128
HD = IN_DIM // 2
CH = 10
NCH = ET // (BB * CH)
NPT = NP // NTILES

_HIGH = jax.lax.Precision.HIGHEST


def _logits_body(xp_ref, w1_ref, a2_ref, sd_ref, m_ref):
    ws = jnp.dot(w1_ref[...], a2_ref[...], precision=_HIGH)
    for g in range(2):
        xg = xp_ref[g]
        s = jnp.sum(xg * ws[:, 0][None, :], axis=1)
        d = jnp.sum(xg * ws[:, 1][None, :], axis=1)
        sd_ref[g, 0] = s
        sd_ref[g, 1] = d
        m = jnp.max(s) + jnp.max(d)
        m = jnp.where(m >= 0, m, 0.2 * m)
        m_ref[g] = jnp.full((16,), m, jnp.float32)


def _logits_call(xp, W1, A2):
    return pl.pallas_call(
        _logits_body,
        out_shape=[
            jax.ShapeDtypeStruct((2, 2, NP), jnp.float32),
            jax.ShapeDtypeStruct((2, 16), jnp.float32),
        ],
    )(xp, W1, A2)


def _sc_body(src_r, dst_r, sd_r, m_r, bidx_r, x_r,
             agg_r, dpart_r, dcomb_r, p_r,
             src_ch, dst_ch, p_ch, s_v, d_v, den_v, tmp_v, comb_v, m_v,
             bidx_v, attn_v, rows_v, rows2_v, agg_sh, sem, sem2):
    pltpu.sync_copy(bidx_r, bidx_v)
    c = lax.axis_index("c")
    t = lax.axis_index("s")
    base = t * NPT

    pltpu.sync_copy(sd_r.at[c, 0], s_v)
    pltpu.sync_copy(sd_r.at[c, 1], d_v)
    pltpu.sync_copy(m_r.at[c], m_v)
    m16 = m_v[...]

    zero16 = jnp.zeros((16,), jnp.float32)

    def _zero_den(i, carry):
        den_v[pl.ds(i * 16, 16)] = zero16
        return carry
    lax.fori_loop(0, NP // 16, _zero_den, 0)

    def _chunk_a(cc, carry):
        pltpu.sync_copy(src_r.at[0, c, t, cc], src_ch)
        pltpu.sync_copy(dst_r.at[c, t, cc], dst_ch)

        def _edge_logits(i, carry2):
            jb = i // (BB // 16)
            k = i % (BB // 16)
            sl = pl.ds(k * 16, 16)
            src16 = src_ch[jb, sl]
            dst16 = dst_ch[jb, sl]
            sg = plsc.load_gather(s_v, [src16 - c * NP])
            dg = plsc.load_gather(d_v, [dst16])
            e = sg + dg
            e = jnp.where(e >= 0, e, 0.2 * e)
            p = jnp.exp(e - m16)
            p_ch[jb, sl] = p
            plsc.addupdate_scatter(den_v, [dst16], p)
            return carry2
        lax.fori_loop(0, CH * (BB // 16), _edge_logits, 0)
        pltpu.sync_copy(p_ch, p_r.at[c, t, cc])
        return carry
    lax.fori_loop(0, NCH, _chunk_a, 0)

    pltpu.sync_copy(den_v, dpart_r.at[c, t])

    def _zero_rows(i, carry):
        rows_v[i // (HD // 16), pl.ds((i % (HD // 16)) * 16, 16)] = zero16
        return carry
    lax.fori_loop(0, BB * (HD // 16), _zero_rows, 0)

    def _zero_agg(i, carry):
        pltpu.sync_copy(rows_v, agg_sh.at[pl.ds(base + i * BB, BB)])
        return carry
    lax.fori_loop(0, NPT // BB, _zero_agg, 0)

    plsc.subcore_barrier()

    pltpu.sync_copy(dpart_r.at[c, 0, pl.ds(base, NPT)], comb_v)
    for tt in range(1, NTILES):
        pltpu.sync_copy(dpart_r.at[c, tt, pl.ds(base, NPT)], tmp_v)

        def _acc(v, carry, _tt=tt):
            sl = pl.ds(v * 16, 16)
            comb_v[sl] = comb_v[sl] + tmp_v[sl]
            return carry
        lax.fori_loop(0, NPT // 16, _acc, 0)
    pltpu.sync_copy(comb_v, dcomb_r.at[c, pl.ds(base, NPT)])

    plsc.subcore_barrier()
    pltpu.sync_copy(dcomb_r.at[c], den_v)

    def _scale_scatter(buf, jj):
        for k in range(BB // 16):
            sl = pl.ds(k * 16, 16)
            p16 = p_ch[jj, sl]
            d16 = plsc.load_gather(den_v, [dst_ch[jj, sl]])
            attn_v[sl] = p16 / (d16 + 1e-16)
        for r in range(BB):
            a16 = plsc.load_gather(attn_v, [bidx_v[r]])
            for q in range(HD // 16):
                sl = pl.ds(q * 16, 16)
                buf[r, sl] = buf[r, sl] * a16
        pltpu.sync_copy(buf, agg_sh.at[dst_ch.at[jj]], add=True)

    for h in range(2):
        def _chunk_c(cc, carry, _h=h):
            pltpu.sync_copy(src_r.at[_h, c, t, cc], src_ch)
            pltpu.sync_copy(dst_r.at[c, t, cc], dst_ch)
            pltpu.sync_copy(p_r.at[c, t, cc], p_ch)

            pltpu.async_copy(x_r.at[src_ch.at[0]], rows_v, sem)

            def _pipe(i, carry2):
                jj0 = 2 * i
                jj1 = 2 * i + 1
                pltpu.async_copy(x_r.at[src_ch.at[jj1]], rows2_v, sem2)
                pltpu.make_async_copy(
                    x_r.at[src_ch.at[jj0]], rows_v, sem).wait()
                _scale_scatter(rows_v, jj0)

                @pl.when(i < CH // 2 - 1)
                def _start_next():
                    pltpu.async_copy(
                        x_r.at[src_ch.at[jj0 + 2]], rows_v, sem)
                pltpu.make_async_copy(
                    x_r.at[src_ch.at[jj1]], rows2_v, sem2).wait()
                _scale_scatter(rows2_v, jj1)
                return carry2
            lax.fori_loop(0, CH // 2, _pipe, 0)
            return carry
        lax.fori_loop(0, NCH, _chunk_c, 0)

        plsc.subcore_barrier()
        pltpu.sync_copy(agg_sh.at[pl.ds(base, NPT)],
                        agg_r.at[h, c, pl.ds(base, NPT)])
        if h == 0:
            def _zero_rows2(i, carry):
                rows_v[i // (HD // 16),
                       pl.ds((i % (HD // 16)) * 16, 16)] = zero16
                return carry
            lax.fori_loop(0, BB * (HD // 16), _zero_rows2, 0)

            def _zero_agg2(i, carry):
                pltpu.sync_copy(rows_v, agg_sh.at[pl.ds(base + i * BB, BB)])
                return carry
            lax.fori_loop(0, NPT // BB, _zero_agg2, 0)
            plsc.subcore_barrier()


def _sc_call(src5, dst5, sd, mv, bidx, xflat):
    mesh = plsc.VectorSubcoreMesh(core_axis_name="c", subcore_axis_name="s")
    kfn = pl.kernel(
        _sc_body,
        out_type=[
            jax.ShapeDtypeStruct((2, 2, NP, HD), jnp.float32),
            jax.ShapeDtypeStruct((2, NTILES, NP), jnp.float32),
            jax.ShapeDtypeStruct((2, NP), jnp.float32),
            jax.ShapeDtypeStruct((2, NTILES, NCH, CH, BB), jnp.float32),
        ],
        mesh=mesh,
        compiler_params=pltpu.CompilerParams(needs_layout_passes=False, use_tc_tiling_on_sc=False),
        scratch_types=[
            pltpu.VMEM((CH, BB), jnp.int32),
            pltpu.VMEM((CH, BB), jnp.int32),
            pltpu.VMEM((CH, BB), jnp.float32),
            pltpu.VMEM((NP,), jnp.float32),
            pltpu.VMEM((NP,), jnp.float32),
            pltpu.VMEM((NP,), jnp.float32),
            pltpu.VMEM((NPT,), jnp.float32),
            pltpu.VMEM((NPT,), jnp.float32),
            pltpu.VMEM((16,), jnp.float32),
            pltpu.VMEM((BB, 16), jnp.int32),
            pltpu.VMEM((BB,), jnp.float32),
            pltpu.VMEM((BB, HD), jnp.float32),
            pltpu.VMEM((BB, HD), jnp.float32),
            pltpu.VMEM_SHARED((NP, HD), jnp.float32),
            pltpu.SemaphoreType.DMA,
            pltpu.SemaphoreType.DMA,
        ],
    )
    return kfn(src5, dst5, sd, mv, bidx, xflat)


def _out_body(agg_ref, w1_ref, w2_ref, z_ref):
    w1 = w1_ref[...]
    v = (jnp.dot(agg_ref[0, 0], w1[:HD], precision=_HIGH)
         + jnp.dot(agg_ref[1, 0], w1[HD:], precision=_HIGH))
    h1 = jnp.where(v > 0, v, jnp.exp(jnp.minimum(v, 0.0)) - 1.0)
    h2 = jnp.dot(h1, w2_ref[...], precision=_HIGH)
    nrm = jnp.sqrt(jnp.sum(h2 * h2, axis=1, keepdims=True))
    z_ref[0] = h2 / (nrm + 1e-12)


def _out_call(agg, W1, W2):
    RB = 1280
    grid = (2, NP // RB)
    return pl.pallas_call(
        _out_body,
        grid=grid,
        in_specs=[
            pl.BlockSpec((2, 1, RB, HD), lambda g, i: (0, g, i, 0)),
            pl.BlockSpec((IN_DIM, HID_DIM), lambda g, i: (0, 0)),
            pl.BlockSpec((HID_DIM, OUT_DIM), lambda g, i: (0, 0)),
        ],
        out_specs=pl.BlockSpec((1, RB, OUT_DIM), lambda g, i: (g, i, 0)),
        out_shape=jax.ShapeDtypeStruct((2, NP, OUT_DIM), jnp.float32),
    )(agg, W1, W2)


def kernel(g_omics1, features_omics1, g_omics2, features_omics2,
           W1, att_src, att_dst, W2):
    pad = NP - N
    xp = jnp.stack([
        jnp.pad(features_omics1, ((0, pad), (0, 0))),
        jnp.pad(features_omics2, ((0, pad), (0, 0))),
    ])
    xflat = xp.reshape(2 * NP, IN_DIM)
    xh = jnp.concatenate([xflat[:, :HD], xflat[:, HD:]], axis=0)
    A2 = jnp.stack([att_src, att_dst], axis=1)

    epad = EP - E
    pad_idx = jnp.full((epad,), NP - 1, jnp.int32)
    src5 = jnp.stack([
        jnp.concatenate([g_omics1[0].astype(jnp.int32), pad_idx]),
        jnp.concatenate([g_omics2[0].astype(jnp.int32), pad_idx]) + NP,
    ]).reshape(2, NTILES, NCH, CH, BB)
    src6 = jnp.stack([src5, src5 + 2 * NP])
    dst5 = jnp.stack([
        jnp.concatenate([g_omics1[1].astype(jnp.int32), pad_idx]),
        jnp.concatenate([g_omics2[1].astype(jnp.int32), pad_idx]),
    ]).reshape(2, NTILES, NCH, CH, BB)

    bidx = jnp.tile(jnp.arange(BB, dtype=jnp.int32)[:, None], (1, 16))
    sd, mv = _logits_call(xp, W1, A2)
    agg, _dp, _dc, _p = _sc_call(src6, dst5, sd, mv, bidx, xh)
    z = _out_call(agg, W1, W2)
    return (z[0, :N], z[1, :N])


# speedup vs baseline: 25.0903x; 1.6884x over previous
import jax
import jax.numpy as jnp
from jax import lax
from jax.experimental import pallas as pl
from jax.experimental.pallas import tpu as pltpu
from jax.experimental.pallas import tpu_sc as plsc

N = 10000
NP = 10240
E = 320000
IN_DIM, HID_DIM, OUT_DIM = 128, 512, 128
NTILES = 16
ET = E // NTILES
BB = 80
HD = IN_DIM // 2
CH = 10
NCH = ET // (BB * CH)
NPT = NP // NTILES

_HIGH = jax.lax.Precision.HIGHEST


def _logits_body(xp_ref, w1_ref, a2_ref, sd_ref, m_ref):
    ws = jnp.dot(w1_ref[...], a2_ref[...], precision=_HIGH)
    for g in range(2):
        xg = xp_ref[g]
        s = jnp.sum(xg * ws[:, 0][None, :], axis=1)
        d = jnp.sum(xg * ws[:, 1][None, :], axis=1)
        sd_ref[g, 0] = s
        sd_ref[g, 1] = d
        m = jnp.max(s) + jnp.max(d)
        m = jnp.where(m >= 0, m, 0.2 * m)
        m_ref[g] = jnp.full((16,), m, jnp.float32)


def _logits_call(xp, W1, A2):
    return pl.pallas_call(
        _logits_body,
        out_shape=[
            jax.ShapeDtypeStruct((2, 2, NP), jnp.float32),
            jax.ShapeDtypeStruct((2, 16), jnp.float32),
        ],
    )(xp, W1, A2)


def _sc_body(src_r, dst_r, sd_r, m_r, bidx_r, x_r,
             agg_r, dpart_r, dcomb_r, p_r,
             src_ch, dst_ch, p_ch, s_v, d_v, den_v, tmp_v, comb_v, m_v,
             bidx_v, attn_v, rows_v, rows2_v, agg_sh, sem, sem2):
    pltpu.sync_copy(bidx_r, bidx_v)
    c = lax.axis_index("c")
    t = lax.axis_index("s")
    base = t * NPT

    pltpu.sync_copy(sd_r.at[c, 0], s_v)
    pltpu.sync_copy(sd_r.at[c, 1], d_v)
    pltpu.sync_copy(m_r.at[c], m_v)
    m16 = m_v[...]

    zero16 = jnp.zeros((16,), jnp.float32)

    def _zero_den(i, carry):
        den_v[pl.ds(i * 16, 16)] = zero16
        return carry
    lax.fori_loop(0, NP // 16, _zero_den, 0)

    def _chunk_a(cc, carry):
        pltpu.sync_copy(src_r.at[0, c, t, cc], src_ch)
        pltpu.sync_copy(dst_r.at[c, t, cc], dst_ch)

        def _edge_logits(i, carry2):
            jb = i // (BB // 16)
            k = i % (BB // 16)
            sl = pl.ds(k * 16, 16)
            src16 = src_ch[jb, sl]
            dst16 = dst_ch[jb, sl]
            sg = plsc.load_gather(s_v, [src16 - c * NP])
            dg = plsc.load_gather(d_v, [dst16])
            e = sg + dg
            e = jnp.where(e >= 0, e, 0.2 * e)
            p = jnp.exp(e - m16)
            p_ch[jb, sl] = p
            plsc.addupdate_scatter(den_v, [dst16], p)
            return carry2
        lax.fori_loop(0, CH * (BB // 16), _edge_logits, 0)
        pltpu.sync_copy(p_ch, p_r.at[c, t, cc])
        return carry
    lax.fori_loop(0, NCH, _chunk_a, 0)

    pltpu.sync_copy(den_v, dpart_r.at[c, t])

    def _zero_rows(i, carry):
        rows_v[i // (HD // 16), pl.ds((i % (HD // 16)) * 16, 16)] = zero16
        return carry
    lax.fori_loop(0, BB * (HD // 16), _zero_rows, 0)

    def _zero_agg(i, carry):
        pltpu.sync_copy(rows_v, agg_sh.at[pl.ds(base + i * BB, BB)])
        return carry
    lax.fori_loop(0, NPT // BB, _zero_agg, 0)

    plsc.subcore_barrier()

    pltpu.sync_copy(dpart_r.at[c, 0, pl.ds(base, NPT)], comb_v)
    for tt in range(1, NTILES):
        pltpu.sync_copy(dpart_r.at[c, tt, pl.ds(base, NPT)], tmp_v)

        def _acc(v, carry, _tt=tt):
            sl = pl.ds(v * 16, 16)
            comb_v[sl] = comb_v[sl] + tmp_v[sl]
            return carry
        lax.fori_loop(0, NPT // 16, _acc, 0)
    pltpu.sync_copy(comb_v, dcomb_r.at[c, pl.ds(base, NPT)])

    plsc.subcore_barrier()
    pltpu.sync_copy(dcomb_r.at[c], den_v)

    def _scale_scatter(buf, jj):
        for k in range(BB // 16):
            sl = pl.ds(k * 16, 16)
            p16 = p_ch[jj, sl]
            d16 = plsc.load_gather(den_v, [dst_ch[jj, sl]])
            attn_v[sl] = p16 / (d16 + 1e-16)
        for k in range(BB // 16):
            at16 = attn_v[pl.ds(k * 16, 16)]
            for r2 in range(16):
                r = k * 16 + r2
                a16 = jnp.full((16,), at16[r2], jnp.float32)
                for q in range(HD // 16):
                    sl = pl.ds(q * 16, 16)
                    buf[r, sl] = buf[r, sl] * a16
        pltpu.sync_copy(buf, agg_sh.at[dst_ch.at[jj]], add=True)

    for h in range(2):
        def _chunk_c(cc, carry, _h=h):
            pltpu.sync_copy(src_r.at[_h, c, t, cc], src_ch)
            pltpu.sync_copy(dst_r.at[c, t, cc], dst_ch)
            pltpu.sync_copy(p_r.at[c, t, cc], p_ch)

            pltpu.async_copy(x_r.at[src_ch.at[0]], rows_v, sem)

            def _pipe(i, carry2):
                jj0 = 2 * i
                jj1 = 2 * i + 1
                pltpu.async_copy(x_r.at[src_ch.at[jj1]], rows2_v, sem2)
                pltpu.make_async_copy(
                    x_r.at[src_ch.at[jj0]], rows_v, sem).wait()
                _scale_scatter(rows_v, jj0)

                @pl.when(i < CH // 2 - 1)
                def _start_next():
                    pltpu.async_copy(
                        x_r.at[src_ch.at[jj0 + 2]], rows_v, sem)
                pltpu.make_async_copy(
                    x_r.at[src_ch.at[jj1]], rows2_v, sem2).wait()
                _scale_scatter(rows2_v, jj1)
                return carry2
            lax.fori_loop(0, CH // 2, _pipe, 0)
            return carry
        lax.fori_loop(0, NCH, _chunk_c, 0)

        plsc.subcore_barrier()
        pltpu.sync_copy(agg_sh.at[pl.ds(base, NPT)],
                        agg_r.at[h, c, pl.ds(base, NPT)])
        if h == 0:
            def _zero_rows2(i, carry):
                rows_v[i // (HD // 16),
                       pl.ds((i % (HD // 16)) * 16, 16)] = zero16
                return carry
            lax.fori_loop(0, BB * (HD // 16), _zero_rows2, 0)

            def _zero_agg2(i, carry):
                pltpu.sync_copy(rows_v, agg_sh.at[pl.ds(base + i * BB, BB)])
                return carry
            lax.fori_loop(0, NPT // BB, _zero_agg2, 0)
            plsc.subcore_barrier()


def _sc_call(src5, dst5, sd, mv, bidx, xflat):
    mesh = plsc.VectorSubcoreMesh(core_axis_name="c", subcore_axis_name="s")
    kfn = pl.kernel(
        _sc_body,
        out_type=[
            jax.ShapeDtypeStruct((2, 2, NP, HD), jnp.float32),
            jax.ShapeDtypeStruct((2, NTILES, NP), jnp.float32),
            jax.ShapeDtypeStruct((2, NP), jnp.float32),
            jax.ShapeDtypeStruct((2, NTILES, NCH, CH, BB), jnp.float32),
        ],
        mesh=mesh,
        compiler_params=pltpu.CompilerParams(needs_layout_passes=False, use_tc_tiling_on_sc=False),
        scratch_types=[
            pltpu.VMEM((CH, BB), jnp.int32),
            pltpu.VMEM((CH, BB), jnp.int32),
            pltpu.VMEM((CH, BB), jnp.float32),
            pltpu.VMEM((NP,), jnp.float32),
            pltpu.VMEM((NP,), jnp.float32),
            pltpu.VMEM((NP,), jnp.float32),
            pltpu.VMEM((NPT,), jnp.float32),
            pltpu.VMEM((NPT,), jnp.float32),
            pltpu.VMEM((16,), jnp.float32),
            pltpu.VMEM((BB, 16), jnp.int32),
            pltpu.VMEM((BB,), jnp.float32),
            pltpu.VMEM((BB, HD), jnp.float32),
            pltpu.VMEM((BB, HD), jnp.float32),
            pltpu.VMEM_SHARED((NP, HD), jnp.float32),
            pltpu.SemaphoreType.DMA,
            pltpu.SemaphoreType.DMA,
        ],
    )
    return kfn(src5, dst5, sd, mv, bidx, xflat)


def _out_body(agg_ref, w1_ref, w2_ref, z_ref):
    w1 = w1_ref[...]
    v = (jnp.dot(agg_ref[0, 0], w1[:HD], precision=_HIGH)
         + jnp.dot(agg_ref[1, 0], w1[HD:], precision=_HIGH))
    h1 = jnp.where(v > 0, v, jnp.exp(jnp.minimum(v, 0.0)) - 1.0)
    h2 = jnp.dot(h1, w2_ref[...], precision=_HIGH)
    nrm = jnp.sqrt(jnp.sum(h2 * h2, axis=1, keepdims=True))
    z_ref[0] = h2 / (nrm + 1e-12)


def _out_call(agg, W1, W2):
    RB = 1280
    grid = (2, NP // RB)
    return pl.pallas_call(
        _out_body,
        grid=grid,
        in_specs=[
            pl.BlockSpec((2, 1, RB, HD), lambda g, i: (0, g, i, 0)),
            pl.BlockSpec((IN_DIM, HID_DIM), lambda g, i: (0, 0)),
            pl.BlockSpec((HID_DIM, OUT_DIM), lambda g, i: (0, 0)),
        ],
        out_specs=pl.BlockSpec((1, RB, OUT_DIM), lambda g, i: (g, i, 0)),
        out_shape=jax.ShapeDtypeStruct((2, NP, OUT_DIM), jnp.float32),
    )(agg, W1, W2)


def kernel(g_omics1, features_omics1, g_omics2, features_omics2,
           W1, att_src, att_dst, W2):
    pad = NP - N
    xp = jnp.stack([
        jnp.pad(features_omics1, ((0, pad), (0, 0))),
        jnp.pad(features_omics2, ((0, pad), (0, 0))),
    ])
    xflat = xp.reshape(2 * NP, IN_DIM)
    xh = jnp.concatenate([xflat[:, :HD], xflat[:, HD:]], axis=0)
    A2 = jnp.stack([att_src, att_dst], axis=1)

    src5 = jnp.stack([g_omics1[0], g_omics2[0] + NP])
    src5 = src5.reshape(2, NTILES, NCH, CH, BB).astype(jnp.int32)
    src6 = jnp.stack([src5, src5 + 2 * NP])
    dst5 = jnp.stack([g_omics1[1], g_omics2[1]])
    dst5 = dst5.reshape(2, NTILES, NCH, CH, BB).astype(jnp.int32)

    bidx = jnp.tile(jnp.arange(BB, dtype=jnp.int32)[:, None], (1, 16))
    sd, mv = _logits_call(xp, W1, A2)
    agg, _dp, _dc, _p = _sc_call(src6, dst5, sd, mv, bidx, xh)
    z = _out_call(agg, W1, W2)
    return (z[0, :N], z[1, :N])
